# Initial kernel scaffold; baseline (speedup 1.0000x reference)
import jax
import jax.numpy as jnp
from jax.experimental import pallas as pl


def kernel(grid, rnd, bt):
    raise NotImplementedError("write your pallas kernel here")


# speedup vs baseline: 2.9099x; 2.9099x over previous
import jax
import jax.numpy as jnp
import numpy as np
from jax import lax
from jax.experimental import pallas as pl
from jax.experimental.pallas import tpu as pltpu
from jax.experimental.pallas import tpu_sc as plsc

X, Y, H = 256, 256, 16
N_COARSE = 16384
BT = 8
NC, NS = 2, 16
NW = NC * NS
PER_W = N_COARSE // NW
CHUNK = 128
NCHUNK = PER_W // CHUNK
SPAN = PER_W * H

SCALE_XY = 102.4
DIST_XY = 51.2
SCALE_H = 8.0
DIST_H = 5.0

_GDN = lax.GatherDimensionNumbers(
    offset_dims=(), collapsed_slice_dims=(0,), start_index_map=(0,))


def _splat(vec, k):
    idx = jnp.full((16, 1), k, jnp.int32)
    return lax.gather(vec, idx, dimension_numbers=_GDN, slice_sizes=(1,),
                      mode=lax.GatherScatterMode.PROMISE_IN_BOUNDS)


def _body(table, rnd3, btzf, btzi, coords_out, idx_out,
          idx_v, xidx_v, yidx_v, rows_x, rows_y, cbuf, ibuf,
          btzf_v, btzi_v, gsem, osem):
    wid = lax.axis_index("s") * NC + lax.axis_index("c")
    pltpu.sync_copy(rnd3.at[wid], idx_v)
    pltpu.sync_copy(btzf, btzf_v)
    pltpu.sync_copy(btzi, btzi_v)
    for k in range(NCHUNK):
        for j in range(CHUNK // 16):
            sl = pl.ds(j * 16, 16)
            two_i = idx_v[k, sl] * 2
            xidx_v[k, sl] = two_i
            yidx_v[k, sl] = two_i + 1
    gathers = []
    for k in range(NCHUNK):
        gathers.append(pltpu.async_copy(
            table.at[xidx_v.at[k]], rows_x.at[pl.ds(k * CHUNK, CHUNK)], gsem))
        gathers.append(pltpu.async_copy(
            table.at[yidx_v.at[k]], rows_y.at[pl.ds(k * CHUNK, CHUNK)], gsem))

    vf = btzf_v[...]
    vi = btzi_v[...]
    lanes = lax.iota(jnp.int32, 16)
    hcoord = lanes.astype(jnp.float32) * (1.0 / (H - 1)) * SCALE_H - DIST_H + vf
    hidx = lanes + vi
    for g in gathers:
        g.wait()

    def body(i, carry):
        xv = rows_x[pl.ds(i * 16, 16)]
        yv = rows_y[pl.ds(i * 16, 16)]
        for k in range(16):
            xs = _splat(xv, k)
            ys = _splat(yv, k)
            sl = pl.ds((i * 16 + k) * H, H)
            cbuf[0, sl] = xs * SCALE_XY - DIST_XY + vf
            cbuf[1, sl] = ys * SCALE_XY - DIST_XY + vf
            cbuf[2, sl] = hcoord
            ibuf[0, sl] = (xs * float(X - 1) + 0.5).astype(jnp.int32) + vi
            ibuf[1, sl] = (ys * float(Y - 1) + 0.5).astype(jnp.int32) + vi
            ibuf[2, sl] = hidx
        return carry

    lax.fori_loop(0, PER_W // 16, body, 0)

    base = wid * SPAN
    copies = []
    for b in range(BT):
        copies.append(pltpu.async_copy(
            cbuf, coords_out.at[b, :, pl.ds(base, SPAN)], osem))
        copies.append(pltpu.async_copy(
            ibuf, idx_out.at[b, :, pl.ds(base, SPAN)], osem))
    for c in copies:
        c.wait()


def kernel(grid, rnd, bt):
    table = grid.reshape(X * Y * 2)
    rnd3 = rnd.reshape(NW, NCHUNK, CHUNK)
    btz = (jnp.asarray(bt) - BT).astype(jnp.int32)
    btzi = jnp.full((16,), btz, jnp.int32)
    btzf = btzi.astype(jnp.float32)

    mesh = plsc.VectorSubcoreMesh(
        core_axis_name="c", subcore_axis_name="s",
        num_cores=NC, num_subcores=NS)
    run = pl.kernel(
        _body,
        out_type=(
            jax.ShapeDtypeStruct((BT, 3, N_COARSE * H), jnp.float32),
            jax.ShapeDtypeStruct((BT, 3, N_COARSE * H), jnp.int32),
        ),
        mesh=mesh,
        scratch_types=[
            pltpu.VMEM((NCHUNK, CHUNK), jnp.int32),
            pltpu.VMEM((NCHUNK, CHUNK), jnp.int32),
            pltpu.VMEM((NCHUNK, CHUNK), jnp.int32),
            pltpu.VMEM((PER_W,), jnp.float32),
            pltpu.VMEM((PER_W,), jnp.float32),
            pltpu.VMEM((3, SPAN), jnp.float32),
            pltpu.VMEM((3, SPAN), jnp.int32),
            pltpu.VMEM((16,), jnp.float32),
            pltpu.VMEM((16,), jnp.int32),
            pltpu.SemaphoreType.DMA,
            pltpu.SemaphoreType.DMA,
        ],
    )
    return run(table, rnd3, btzf, btzi)
